# Initial kernel scaffold; baseline (speedup 1.0000x reference)
import jax
import jax.numpy as jnp
from jax.experimental import pallas as pl
from jax.experimental.pallas import tpu as pltpu

N = 4096
HID = 1024
FFNN = 3072
K = 1638
LN_EPS = 1e-5
ROWS = 512
NG = 512
NSLOT = 13 * 128


def _safe_log(x):
    return jnp.clip(jnp.log(jnp.clip(x, 1e-12, 1.0)), -100.0, 0.0)


def _mlp_kernel(x_ref, w_ref, b_ref, g_ref, bb_ref, wc_ref, bc_ref, out_ref):
    h = jnp.dot(x_ref[...], w_ref[...], preferred_element_type=jnp.float32)
    h = h + b_ref[...]
    h = jax.nn.gelu(h, approximate=False)
    mu = jnp.mean(h, axis=-1, keepdims=True)
    var = jnp.mean((h - mu) ** 2, axis=-1, keepdims=True)
    h = (h - mu) / jnp.sqrt(var + LN_EPS) * g_ref[...] + bb_ref[...]
    logits = jnp.dot(h, wc_ref[...], preferred_element_type=jnp.float32)
    logits = logits + bc_ref[...]
    out_ref[...] = jax.nn.sigmoid(logits)


def _flat_cumsum(x):
    li = jax.lax.broadcasted_iota(jnp.int32, (128, 128), 0)
    lj = jax.lax.broadcasted_iota(jnp.int32, (128, 128), 1)
    upper = (li <= lj).astype(jnp.float32)
    within = jnp.dot(x, upper, preferred_element_type=jnp.float32)
    row_tot = within[:, 127:128]
    ri = jax.lax.broadcasted_iota(jnp.int32, (32, 32), 0)
    rj = jax.lax.broadcasted_iota(jnp.int32, (32, 32), 1)
    strict = (rj < ri).astype(jnp.float32)
    offs = jnp.dot(strict, row_tot, preferred_element_type=jnp.float32)
    return within + offs


def _select_kernel(p_ref, q_ref, l1m_ref, tot_ref):
    p = p_ref[...]
    bits = jax.lax.bitcast_convert_type(p, jnp.int32)

    def body(_, carry):
        lo, hi = carry
        mid = lo + (hi - lo + 1) // 2
        cnt = jnp.sum((bits >= mid).astype(jnp.int32))
        ok = cnt >= K
        return jnp.where(ok, mid, lo), jnp.where(ok, hi, mid - 1)

    thr, _ = jax.lax.fori_loop(
        0, 31, body, (jnp.int32(0), jnp.int32(0x3F800000))
    )
    gt = bits > thr
    eq = bits == thr
    n_gt = jnp.sum(gt.astype(jnp.float32))
    need = jnp.float32(K) - n_gt
    cs_eq = _flat_cumsum(eq.astype(jnp.float32))
    sel = jnp.logical_or(gt, jnp.logical_and(eq, cs_eq <= need))
    pos = _flat_cumsum(sel.astype(jnp.float32))
    q_ref[...] = jnp.where(sel, pos - 1.0, 8190.0).astype(jnp.int32)
    l1m = _safe_log(1.0 - p)
    l1m_ref[...] = l1m
    tot_ref[0, 0] = jnp.sum(l1m)


def _compact_kernel(pc_ref, qc_ref, gold_ref, l1m_ref, tot_ref,
                    st_ref, it_ref, cost_ref):
    probs_col = pc_ref[...]
    q_col = qc_ref[...]
    iota_col = jax.lax.broadcasted_iota(jnp.float32, (N, 1), 0)
    hi = jax.lax.Precision.HIGHEST
    for r in range(NSLOT // 128):
        lane = jax.lax.broadcasted_iota(jnp.int32, (N, 128), 1) + r * 128
        oh = (q_col == lane).astype(jnp.float32)
        s_col = jax.lax.dot_general(
            oh, probs_col, (((0,), (0,)), ((), ())), precision=hi,
            preferred_element_type=jnp.float32)
        i_col = jax.lax.dot_general(
            oh, iota_col, (((0,), (0,)), ((), ())), precision=hi,
            preferred_element_type=jnp.float32)
        st_ref[:, r:r + 1] = s_col
        it_ref[:, r:r + 1] = jnp.round(i_col).astype(jnp.int32)
    gcol = gold_ref[...]
    gl = jax.lax.broadcasted_iota(jnp.int32, (NG, N), 1)
    ohg = (gcol == gl).astype(jnp.float32)
    gp = jnp.dot(ohg, probs_col, precision=hi,
                 preferred_element_type=jnp.float32)
    cost_gold = -jnp.mean(_safe_log(gp))
    counts = jnp.dot(jnp.ones((1, NG), jnp.float32), ohg,
                     preferred_element_type=jnp.float32)
    mask_f = (counts > 0.5).astype(jnp.float32)
    masked = jnp.dot(mask_f, l1m_ref[...], precision=hi,
                     preferred_element_type=jnp.float32)
    junk_count = jnp.float32(N) - jnp.sum(mask_f)
    junk_sum = tot_ref[0, 0] - masked[0, 0]
    cost_ref[0, 0] = cost_gold - junk_sum / junk_count


def kernel(mentions, gold_indices, W_dense, b_dense, ln_gamma, ln_beta,
           W_cls, b_cls):
    probs_col = pl.pallas_call(
        _mlp_kernel,
        grid=(N // ROWS,),
        in_specs=[
            pl.BlockSpec((ROWS, HID), lambda i: (i, 0)),
            pl.BlockSpec((HID, FFNN), lambda i: (0, 0)),
            pl.BlockSpec((1, FFNN), lambda i: (0, 0)),
            pl.BlockSpec((1, FFNN), lambda i: (0, 0)),
            pl.BlockSpec((FFNN, 1), lambda i: (0, 0)),
            pl.BlockSpec((1, 1), lambda i: (0, 0)),
        ],
        out_specs=pl.BlockSpec((ROWS, 1), lambda i: (i, 0)),
        out_shape=jax.ShapeDtypeStruct((N, 1), jnp.float32),
    )(
        mentions,
        W_dense,
        b_dense.reshape(1, FFNN),
        ln_gamma.reshape(1, FFNN),
        ln_beta.reshape(1, FFNN),
        W_cls,
        b_cls.reshape(1, 1),
    )

    probs32 = probs_col.reshape(32, 128)
    q32, l1m32, tot = pl.pallas_call(
        _select_kernel,
        out_shape=(
            jax.ShapeDtypeStruct((32, 128), jnp.int32),
            jax.ShapeDtypeStruct((32, 128), jnp.float32),
            jax.ShapeDtypeStruct((1, 1), jnp.float32),
        ),
    )(probs32)

    scoresT, idxT, cost = pl.pallas_call(
        _compact_kernel,
        out_shape=(
            jax.ShapeDtypeStruct((128, NSLOT // 128), jnp.float32),
            jax.ShapeDtypeStruct((128, NSLOT // 128), jnp.int32),
            jax.ShapeDtypeStruct((1, 1), jnp.float32),
        ),
    )(
        probs_col,
        q32.reshape(N, 1),
        gold_indices.astype(jnp.int32).reshape(NG, 1),
        l1m32.reshape(N, 1),
        tot,
    )

    top_scores = scoresT.T.reshape(-1)[:K]
    indices = idxT.T.reshape(-1)[:K]
    return (top_scores, indices, cost.reshape(()))


# speedup vs baseline: 1.6270x; 1.6270x over previous
import jax
import jax.numpy as jnp
from jax.experimental import pallas as pl
from jax.experimental.pallas import tpu as pltpu

N = 4096
HID = 1024
FFNN = 3072
K = 1638
LN_EPS = 1e-5
ROWS = 512
NG = 512
NSLOT = 13 * 128


def _safe_log(x):
    return jnp.clip(jnp.log(jnp.clip(x, 1e-12, 1.0)), -100.0, 0.0)


def _mlp_kernel(x_ref, w_ref, b_ref, g_ref, bb_ref, wc_ref, bc_ref, out_ref):
    h = jnp.dot(x_ref[...], w_ref[...], preferred_element_type=jnp.float32)
    h = h + b_ref[...]
    h = 0.5 * h * (1.0 + jax.lax.erf(h * 0.7071067811865476))
    mu = jnp.mean(h, axis=-1, keepdims=True)
    var = jnp.mean((h - mu) ** 2, axis=-1, keepdims=True)
    h = (h - mu) / jnp.sqrt(var + LN_EPS) * g_ref[...] + bb_ref[...]
    logits = jnp.dot(h, wc_ref[...], preferred_element_type=jnp.float32)
    logits = logits + bc_ref[...]
    out_ref[...] = jax.nn.sigmoid(logits)


def _flat_cumsum(x):
    li = jax.lax.broadcasted_iota(jnp.int32, (128, 128), 0)
    lj = jax.lax.broadcasted_iota(jnp.int32, (128, 128), 1)
    upper = (li <= lj).astype(jnp.float32)
    within = jnp.dot(x, upper, preferred_element_type=jnp.float32)
    row_tot = within[:, 127:128]
    ri = jax.lax.broadcasted_iota(jnp.int32, (32, 32), 0)
    rj = jax.lax.broadcasted_iota(jnp.int32, (32, 32), 1)
    strict = (rj < ri).astype(jnp.float32)
    offs = jnp.dot(strict, row_tot, preferred_element_type=jnp.float32)
    return within + offs


def _select_kernel(p_ref, q_ref, l1m_ref, tot_ref):
    p = p_ref[...]
    bits = jax.lax.bitcast_convert_type(p, jnp.int32)

    def body(_, carry):
        lo, hi = carry
        mid = lo + (hi - lo + 1) // 2
        cnt = jnp.sum((bits >= mid).astype(jnp.int32))
        ok = cnt >= K
        return jnp.where(ok, mid, lo), jnp.where(ok, hi, mid - 1)

    thr, _ = jax.lax.fori_loop(
        0, 31, body, (jnp.int32(0), jnp.int32(0x3F800000))
    )
    gt = bits > thr
    eq = bits == thr
    n_gt = jnp.sum(gt.astype(jnp.float32))
    need = jnp.float32(K) - n_gt
    cs_eq = _flat_cumsum(eq.astype(jnp.float32))
    sel = jnp.logical_or(gt, jnp.logical_and(eq, cs_eq <= need))
    pos = _flat_cumsum(sel.astype(jnp.float32))
    q_ref[...] = jnp.where(sel, pos - 1.0, 8190.0).astype(jnp.int32)
    l1m = _safe_log(1.0 - p)
    l1m_ref[...] = l1m
    tot_ref[...] = jnp.sum(l1m).reshape(1, 1)


def _compact_kernel(pc_ref, qc_ref, gold_ref, l1m_ref, tot_ref,
                    st_ref, it_ref, cost_ref):
    probs_col = pc_ref[...]
    q_col = qc_ref[...]
    iota_col = jax.lax.broadcasted_iota(jnp.int32, (N, 1), 0).astype(
        jnp.float32)
    hi = jax.lax.Precision.HIGHEST
    for r in range(NSLOT // 128):
        lane = jax.lax.broadcasted_iota(jnp.int32, (N, 128), 1) + r * 128
        oh = (q_col == lane).astype(jnp.float32)
        s_col = jax.lax.dot_general(
            oh, probs_col, (((0,), (0,)), ((), ())), precision=hi,
            preferred_element_type=jnp.float32)
        i_col = jax.lax.dot_general(
            oh, iota_col, (((0,), (0,)), ((), ())), precision=hi,
            preferred_element_type=jnp.float32)
        st_ref[:, r:r + 1] = s_col
        it_ref[:, r:r + 1] = jnp.round(i_col).astype(jnp.int32)
    gcol = gold_ref[...]
    gl = jax.lax.broadcasted_iota(jnp.int32, (NG, N), 1)
    ohg = (gcol == gl).astype(jnp.float32)
    gp = jnp.dot(ohg, probs_col, precision=hi,
                 preferred_element_type=jnp.float32)
    cost_gold = -jnp.mean(_safe_log(gp))
    counts = jnp.dot(jnp.ones((1, NG), jnp.float32), ohg,
                     preferred_element_type=jnp.float32)
    mask_f = (counts > 0.5).astype(jnp.float32)
    masked = jnp.dot(mask_f, l1m_ref[...], precision=hi,
                     preferred_element_type=jnp.float32)
    junk_count = jnp.float32(N) - jnp.sum(mask_f)
    junk_sum = tot_ref[...] - masked
    cost_ref[...] = cost_gold.reshape(1, 1) - junk_sum / junk_count


def kernel(mentions, gold_indices, W_dense, b_dense, ln_gamma, ln_beta,
           W_cls, b_cls):
    probs_col = pl.pallas_call(
        _mlp_kernel,
        grid=(N // ROWS,),
        in_specs=[
            pl.BlockSpec((ROWS, HID), lambda i: (i, 0)),
            pl.BlockSpec((HID, FFNN), lambda i: (0, 0)),
            pl.BlockSpec((1, FFNN), lambda i: (0, 0)),
            pl.BlockSpec((1, FFNN), lambda i: (0, 0)),
            pl.BlockSpec((1, FFNN), lambda i: (0, 0)),
            pl.BlockSpec((FFNN, 1), lambda i: (0, 0)),
            pl.BlockSpec((1, 1), lambda i: (0, 0)),
        ],
        out_specs=pl.BlockSpec((ROWS, 1), lambda i: (i, 0)),
        out_shape=jax.ShapeDtypeStruct((N, 1), jnp.float32),
    )(
        mentions,
        W_dense,
        b_dense.reshape(1, FFNN),
        ln_gamma.reshape(1, FFNN),
        ln_beta.reshape(1, FFNN),
        W_cls,
        b_cls.reshape(1, 1),
    )

    probs32 = probs_col.reshape(32, 128)
    q32, l1m32, tot = pl.pallas_call(
        _select_kernel,
        out_shape=(
            jax.ShapeDtypeStruct((32, 128), jnp.int32),
            jax.ShapeDtypeStruct((32, 128), jnp.float32),
            jax.ShapeDtypeStruct((1, 1), jnp.float32),
        ),
    )(probs32)

    scoresT, idxT, cost = pl.pallas_call(
        _compact_kernel,
        out_shape=(
            jax.ShapeDtypeStruct((128, NSLOT // 128), jnp.float32),
            jax.ShapeDtypeStruct((128, NSLOT // 128), jnp.int32),
            jax.ShapeDtypeStruct((1, 1), jnp.float32),
        ),
    )(
        probs_col,
        q32.reshape(N, 1),
        gold_indices.astype(jnp.int32).reshape(NG, 1),
        l1m32.reshape(N, 1),
        tot,
    )

    top_scores = scoresT.T.reshape(-1)[:K]
    indices = idxT.T.reshape(-1)[:K]
    return (top_scores, indices, cost.reshape(()))


# speedup vs baseline: 2.4186x; 1.4865x over previous
import jax
import jax.numpy as jnp
from jax.experimental import pallas as pl
from jax.experimental.pallas import tpu as pltpu

N = 4096
HID = 1024
FFNN = 3072
K = 1638
LN_EPS = 1e-5
ROWS = 512
NG = 512
NSLOT = 13 * 128


def _safe_log(x):
    return jnp.clip(jnp.log(jnp.clip(x, 1e-12, 1.0)), -100.0, 0.0)


def _mlp_kernel(x_ref, w_ref, b_ref, g_ref, bb_ref, wc_ref, bc_ref, out_ref):
    h = jnp.dot(x_ref[...], w_ref[...], preferred_element_type=jnp.float32)
    h = h + b_ref[...]
    h = 0.5 * h * (1.0 + jax.lax.erf(h * 0.7071067811865476))
    mu = jnp.mean(h, axis=-1, keepdims=True)
    var = jnp.mean((h - mu) ** 2, axis=-1, keepdims=True)
    h = (h - mu) / jnp.sqrt(var + LN_EPS) * g_ref[...] + bb_ref[...]
    logits = jnp.dot(h, wc_ref[...], preferred_element_type=jnp.float32)
    logits = logits + bc_ref[...]
    out_ref[...] = jax.nn.sigmoid(logits)


def _flat_cumsum(x):
    li = jax.lax.broadcasted_iota(jnp.int32, (128, 128), 0)
    lj = jax.lax.broadcasted_iota(jnp.int32, (128, 128), 1)
    upper = (li <= lj).astype(jnp.float32)
    within = jnp.dot(x, upper, preferred_element_type=jnp.float32)
    row_tot = within[:, 127:128]
    ri = jax.lax.broadcasted_iota(jnp.int32, (32, 32), 0)
    rj = jax.lax.broadcasted_iota(jnp.int32, (32, 32), 1)
    strict = (rj < ri).astype(jnp.float32)
    offs = jnp.dot(strict, row_tot, preferred_element_type=jnp.float32)
    return within + offs


def _select_kernel(p_ref, q_ref, l1m_ref, tot_ref):
    p = p_ref[...]
    bits = jax.lax.bitcast_convert_type(p, jnp.int32)

    def body(_, carry):
        lo, hi = carry
        mid = lo + (hi - lo + 1) // 2
        cnt = jnp.sum((bits >= mid).astype(jnp.int32))
        ok = cnt >= K
        return jnp.where(ok, mid, lo), jnp.where(ok, hi, mid - 1)

    thr, _ = jax.lax.fori_loop(
        0, 31, body, (jnp.int32(0), jnp.int32(0x3F800000))
    )
    gt = bits > thr
    eq = bits == thr
    n_gt = jnp.sum(gt.astype(jnp.float32))
    need = jnp.float32(K) - n_gt
    cs_eq = _flat_cumsum(eq.astype(jnp.float32))
    sel = jnp.logical_or(gt, jnp.logical_and(eq, cs_eq <= need))
    pos = _flat_cumsum(sel.astype(jnp.float32))
    q_ref[...] = jnp.where(sel, pos - 1.0, 8190.0).astype(jnp.int32)
    l1m = _safe_log(1.0 - p)
    l1m_ref[...] = l1m
    tot_ref[...] = jnp.sum(l1m).reshape(1, 1)


def _compact_kernel(pc_ref, qc_ref, gold_ref, l1m_ref, tot_ref,
                    s_ref, i_ref, cost_ref):
    probs_col = pc_ref[...]
    q_col = qc_ref[...]
    phi = probs_col.astype(jnp.bfloat16)
    plo = (probs_col - phi.astype(jnp.float32)).astype(jnp.bfloat16)
    ii = jax.lax.broadcasted_iota(jnp.int32, (N, 1), 0)
    ihi = (ii // 128).astype(jnp.bfloat16)
    ilo = (ii % 128).astype(jnp.bfloat16)
    vals = jnp.concatenate([phi, plo, ihi, ilo], axis=1)
    for r in range(NSLOT // 128):
        lane = jax.lax.broadcasted_iota(jnp.int32, (N, 128), 1) + r * 128
        oh = (q_col == lane).astype(jnp.float32).astype(jnp.bfloat16)
        blk = jax.lax.dot_general(
            oh, vals, (((0,), (0,)), ((), ())),
            preferred_element_type=jnp.float32)
        s_ref[pl.ds(r * 128, 128), :] = blk[:, 0:1] + blk[:, 1:2]
        i_ref[pl.ds(r * 128, 128), :] = jnp.round(
            blk[:, 2:3] * 128.0 + blk[:, 3:4]).astype(jnp.int32)
    gcol = gold_ref[...]
    gl = jax.lax.broadcasted_iota(jnp.int32, (NG, N), 1)
    ohg = (gcol == gl).astype(jnp.float32).astype(jnp.bfloat16)
    pv = jnp.concatenate([phi, plo], axis=1)
    gp2 = jnp.dot(ohg, pv, preferred_element_type=jnp.float32)
    gp = gp2[:, 0:1] + gp2[:, 1:2]
    cost_gold = -jnp.mean(_safe_log(gp))
    counts = jnp.dot(jnp.ones((1, NG), jnp.bfloat16), ohg,
                     preferred_element_type=jnp.float32)
    mask_f = (counts > 0.5).astype(jnp.float32)
    l1hi = l1m_ref[...].astype(jnp.bfloat16)
    l1lo = (l1m_ref[...] - l1hi.astype(jnp.float32)).astype(jnp.bfloat16)
    l1v = jnp.concatenate([l1hi, l1lo], axis=1)
    masked2 = jnp.dot(mask_f.astype(jnp.bfloat16), l1v,
                      preferred_element_type=jnp.float32)
    masked = masked2[:, 0:1] + masked2[:, 1:2]
    junk_count = jnp.float32(N) - jnp.sum(mask_f)
    junk_sum = tot_ref[...] - masked
    cost_ref[...] = cost_gold.reshape(1, 1) - junk_sum / junk_count


def kernel(mentions, gold_indices, W_dense, b_dense, ln_gamma, ln_beta,
           W_cls, b_cls):
    probs_col = pl.pallas_call(
        _mlp_kernel,
        grid=(N // ROWS,),
        in_specs=[
            pl.BlockSpec((ROWS, HID), lambda i: (i, 0)),
            pl.BlockSpec((HID, FFNN), lambda i: (0, 0)),
            pl.BlockSpec((1, FFNN), lambda i: (0, 0)),
            pl.BlockSpec((1, FFNN), lambda i: (0, 0)),
            pl.BlockSpec((1, FFNN), lambda i: (0, 0)),
            pl.BlockSpec((FFNN, 1), lambda i: (0, 0)),
            pl.BlockSpec((1, 1), lambda i: (0, 0)),
        ],
        out_specs=pl.BlockSpec((ROWS, 1), lambda i: (i, 0)),
        out_shape=jax.ShapeDtypeStruct((N, 1), jnp.float32),
    )(
        mentions,
        W_dense,
        b_dense.reshape(1, FFNN),
        ln_gamma.reshape(1, FFNN),
        ln_beta.reshape(1, FFNN),
        W_cls,
        b_cls.reshape(1, 1),
    )

    probs32 = probs_col.reshape(32, 128)
    q32, l1m32, tot = pl.pallas_call(
        _select_kernel,
        out_shape=(
            jax.ShapeDtypeStruct((32, 128), jnp.int32),
            jax.ShapeDtypeStruct((32, 128), jnp.float32),
            jax.ShapeDtypeStruct((1, 1), jnp.float32),
        ),
    )(probs32)

    s_col, i_col, cost = pl.pallas_call(
        _compact_kernel,
        out_shape=(
            jax.ShapeDtypeStruct((NSLOT, 1), jnp.float32),
            jax.ShapeDtypeStruct((NSLOT, 1), jnp.int32),
            jax.ShapeDtypeStruct((1, 1), jnp.float32),
        ),
    )(
        probs_col,
        q32.reshape(N, 1),
        gold_indices.astype(jnp.int32).reshape(NG, 1),
        l1m32.reshape(N, 1),
        tot,
    )

    top_scores = s_col.reshape(-1)[:K]
    indices = i_col.reshape(-1)[:K]
    return (top_scores, indices, cost.reshape(()))
